# Initial kernel scaffold; baseline (speedup 1.0000x reference)
import jax
import jax.numpy as jnp
from jax.experimental import pallas as pl


def kernel(x, s, centroids, sn_weight, sn_bias, running_sn_weight):
    raise NotImplementedError("write your pallas kernel here")


# speedup vs baseline: 3.4637x; 3.4637x over previous
import jax
import jax.numpy as jnp
from jax.experimental import pallas as pl

_NC = 64


def _body(x_ref, s_ref, ct_ref, snw_ref, snb_ref, rwt_ref, o_ref):
    xb = x_ref[:]
    ct = ct_ref[:]
    prod = jnp.dot(xb, ct, preferred_element_type=jnp.float32)
    cn = jnp.sum(ct * ct, axis=0, keepdims=True)
    xx = jnp.sum(xb * xb, axis=1, keepdims=True)
    d2 = (xx - 2.0 * prod) + cn

    r = xb.shape[0]
    iota = jax.lax.broadcasted_iota(jnp.int32, (r, _NC), 1)
    minv = jnp.min(d2, axis=1, keepdims=True)
    cand = jnp.where(d2 == minv, iota, _NC)
    cl = jnp.min(cand, axis=1, keepdims=True)
    onehot = (iota == cl).astype(jnp.float32)

    wabs = jnp.abs(rwt_ref[:])
    denom = jnp.maximum(wabs[0:1, :] + wabs[1:2, :], 1e-12)
    wn = wabs / denom
    coef_a = jnp.sum(snw_ref[:] * wn, axis=0, keepdims=True)
    coef_b = jnp.sum(snb_ref[:] * wn, axis=0, keepdims=True)

    ga = jnp.sum(onehot * coef_a, axis=1, keepdims=True)
    gb = jnp.sum(onehot * coef_b, axis=1, keepdims=True)
    o_ref[:] = s_ref[:] * ga + gb


@jax.jit
def kernel(x, s, centroids, sn_weight, sn_bias, running_sn_weight):
    n, d = x.shape
    r = 512
    out = pl.pallas_call(
        _body,
        grid=(n // r,),
        in_specs=[
            pl.BlockSpec((r, d), lambda i: (i, 0)),
            pl.BlockSpec((r, 1), lambda i: (i, 0)),
            pl.BlockSpec((d, _NC), lambda i: (0, 0)),
            pl.BlockSpec((2, _NC), lambda i: (0, 0)),
            pl.BlockSpec((2, _NC), lambda i: (0, 0)),
            pl.BlockSpec((2, _NC), lambda i: (0, 0)),
        ],
        out_specs=pl.BlockSpec((r, 1), lambda i: (i, 0)),
        out_shape=jax.ShapeDtypeStruct((n, 1), jnp.float32),
    )(x, s.reshape(n, 1), centroids.T, sn_weight, sn_bias, running_sn_weight.T)
    return out


# speedup vs baseline: 3.7682x; 1.0879x over previous
import jax
import jax.numpy as jnp
from jax.experimental import pallas as pl

_NC = 64


def _body(x_ref, s_ref, ct_ref, snw_ref, snb_ref, rwt_ref, o_ref):
    xb = x_ref[:]
    ct = ct_ref[:]
    prod = jnp.dot(xb, ct, preferred_element_type=jnp.float32)
    cn = jnp.sum(ct * ct, axis=0, keepdims=True)
    xx = jnp.sum(xb * xb, axis=1, keepdims=True)
    d2 = (xx - 2.0 * prod) + cn

    r = xb.shape[0]
    iota = jax.lax.broadcasted_iota(jnp.int32, (r, _NC), 1)
    minv = jnp.min(d2, axis=1, keepdims=True)
    cand = jnp.where(d2 == minv, iota, _NC)
    cl = jnp.min(cand, axis=1, keepdims=True)
    onehot = (iota == cl).astype(jnp.float32)

    wabs = jnp.abs(rwt_ref[:])
    denom = jnp.maximum(wabs[0:1, :] + wabs[1:2, :], 1e-12)
    wn = wabs / denom
    coef_a = jnp.sum(snw_ref[:] * wn, axis=0, keepdims=True)
    coef_b = jnp.sum(snb_ref[:] * wn, axis=0, keepdims=True)

    ga = jnp.sum(onehot * coef_a, axis=1, keepdims=True)
    gb = jnp.sum(onehot * coef_b, axis=1, keepdims=True)
    o_ref[:] = s_ref[:] * ga + gb


@jax.jit
def kernel(x, s, centroids, sn_weight, sn_bias, running_sn_weight):
    n, d = x.shape
    r = 1024
    out = pl.pallas_call(
        _body,
        grid=(n // r,),
        in_specs=[
            pl.BlockSpec((r, d), lambda i: (i, 0)),
            pl.BlockSpec((r, 1), lambda i: (i, 0)),
            pl.BlockSpec((d, _NC), lambda i: (0, 0)),
            pl.BlockSpec((2, _NC), lambda i: (0, 0)),
            pl.BlockSpec((2, _NC), lambda i: (0, 0)),
            pl.BlockSpec((2, _NC), lambda i: (0, 0)),
        ],
        out_specs=pl.BlockSpec((r, 1), lambda i: (i, 0)),
        out_shape=jax.ShapeDtypeStruct((n, 1), jnp.float32),
    )(x, s.reshape(n, 1), centroids.T, sn_weight, sn_bias, running_sn_weight.T)
    return out
